# baseline (speedup 1.0000x reference)
import functools

import jax
import jax.numpy as jnp
from jax import lax
from jax.experimental import pallas as pl
from jax.experimental.pallas import tpu as pltpu
from jax.experimental.pallas import tpu_sc as plsc

N = 10000
E = 320000
FEAT_LEN = 150
FEAT_PAD = 152
DIM = 128
NUM_GRAPHS = 64

NC = 2
NS = 16
NW = NC * NS
EPT = E // NW
NP = 10240
ROWS_PER_SUB = NP // NS

_mesh = plsc.VectorSubcoreMesh(core_axis_name="c", subcore_axis_name="s")


_DEG_CH = 2000


@functools.partial(
    pl.kernel,
    mesh=_mesh,
    out_type=[
        jax.ShapeDtypeStruct((NW, 1, N), jnp.float32),
        jax.ShapeDtypeStruct((NW, 1, N), jnp.float32),
    ],
    scratch_types=[
        pltpu.VMEM((_DEG_CH,), jnp.int32),
        pltpu.VMEM((_DEG_CH,), jnp.int32),
        pltpu.VMEM((N,), jnp.float32),
        pltpu.VMEM((N,), jnp.float32),
    ],
    compiler_params=pltpu.CompilerParams(needs_layout_passes=False),
)
def _deg_kernel(src_hbm, dst_hbm, osrc_hbm, odst_hbm, sidx, didx, scnt, dcnt):
    c = lax.axis_index("c")
    s = lax.axis_index("s")
    wid = s * NC + c
    base = wid * EPT

    zeros16 = jnp.zeros((16,), jnp.float32)

    def zero_body(i, carry):
        scnt[pl.ds(i * 16, 16)] = zeros16
        dcnt[pl.ds(i * 16, 16)] = zeros16
        return carry

    lax.fori_loop(0, N // 16, zero_body, 0)

    ones16 = jnp.ones((16,), jnp.float32)

    def chunk_body(k, carry):
        off = base + k * _DEG_CH
        pltpu.sync_copy(src_hbm.at[pl.ds(off, _DEG_CH)], sidx)
        pltpu.sync_copy(dst_hbm.at[pl.ds(off, _DEG_CH)], didx)

        def vec_body(i, inner):
            si = sidx[pl.ds(i * 16, 16)]
            plsc.addupdate_scatter(scnt, [si], ones16)
            di = didx[pl.ds(i * 16, 16)]
            plsc.addupdate_scatter(dcnt, [di], ones16)
            return inner

        lax.fori_loop(0, _DEG_CH // 16, vec_body, 0)
        return carry

    lax.fori_loop(0, EPT // _DEG_CH, chunk_body, 0)

    pltpu.sync_copy(scnt, osrc_hbm.at[wid, 0])
    pltpu.sync_copy(dcnt, odst_hbm.at[wid, 0])


_AGG_CH = 80
_NBUF = 4
_GROUP = _NBUF * _AGG_CH
_NGRP = EPT // _GROUP
_TAIL = EPT - _NGRP * _GROUP


@functools.partial(
    pl.kernel,
    mesh=_mesh,
    out_type=jax.ShapeDtypeStruct((NC, NP, DIM), jnp.float32),
    scratch_types=[
        pltpu.VMEM((_GROUP,), jnp.int32),
        pltpu.VMEM((_GROUP,), jnp.int32),
        pltpu.VMEM((_GROUP,), jnp.int32),
        pltpu.VMEM((_GROUP,), jnp.int32),
    ] + [pltpu.VMEM((_AGG_CH,), jnp.int32) for _ in range(2 * _NBUF)]
    + [
        pltpu.VMEM((_NBUF, _AGG_CH, DIM), jnp.float32),
        pltpu.VMEM((32, DIM), jnp.float32),
        pltpu.VMEM_SHARED((NP, DIM), jnp.float32),
        pltpu.SemaphoreType.DMA,
    ]
    + [pltpu.SemaphoreType.DMA for _ in range(2 * _NBUF)],
    compiler_params=pltpu.CompilerParams(needs_layout_passes=False),
)
def _agg_kernel(x_hbm, src_hbm, dst_hbm, out_hbm, sflat0, sflat1,
                dflat0, dflat1,
                d00, d01, d02, d03, d10, d11, d12, d13,
                rows, zbuf, acc, isem,
                gsem0, gsem1, gsem2, gsem3, ssem0, ssem1, ssem2, ssem3):
    c = lax.axis_index("c")
    s = lax.axis_index("s")
    wid = s * NC + c
    base = wid * EPT
    sflat = (sflat0, sflat1)
    dflat = (dflat0, dflat1)
    didx = ((d00, d01, d02, d03), (d10, d11, d12, d13))
    gsem = (gsem0, gsem1, gsem2, gsem3)
    ssem = (ssem0, ssem1, ssem2, ssem3)

    zeros16 = jnp.zeros((16,), jnp.float32)

    def zfill(i, carry):
        def zlane(j, inner):
            zbuf[i, pl.ds(j * 16, 16)] = zeros16
            return inner

        return lax.fori_loop(0, DIM // 16, zlane, carry)

    lax.fori_loop(0, 32, zfill, 0)

    def zcopy(t, carry):
        pltpu.async_copy(zbuf, acc.at[pl.ds(s * ROWS_PER_SUB + t * 32, 32)], isem)
        return carry

    lax.fori_loop(0, ROWS_PER_SUB // 32, zcopy, 0)

    def zdrain(t, carry):
        pltpu.make_async_copy(zbuf, acc.at[pl.ds(0, 32)], isem).wait()
        return carry

    lax.fori_loop(0, ROWS_PER_SUB // 32, zdrain, 0)
    plsc.subcore_barrier()

    def load_idx_group(g, p):
        off = base + g * _GROUP
        pltpu.async_copy(src_hbm.at[pl.ds(off, _GROUP)], sflat[p], isem)
        pltpu.async_copy(dst_hbm.at[pl.ds(off, _GROUP)], dflat[p], isem)

    def wait_idx(p):
        pltpu.make_async_copy(src_hbm.at[pl.ds(0, _GROUP)], sflat[p], isem).wait()
        pltpu.make_async_copy(dst_hbm.at[pl.ds(0, _GROUP)], dflat[p], isem).wait()

    def spread_didx(p):
        for b in range(_NBUF):
            for j in range(_AGG_CH // 16):
                didx[p][b][pl.ds(j * 16, 16)] = (
                    dflat[p][pl.ds(b * _AGG_CH + j * 16, 16)])

    def gather_start(p, b):
        pltpu.async_copy(
            x_hbm.at[sflat[p].at[pl.ds(b * _AGG_CH, _AGG_CH)]], rows.at[b], gsem[b])

    def gather_wait(b):
        pltpu.make_async_copy(
            x_hbm.at[sflat[0].at[pl.ds(0, _AGG_CH)]], rows.at[b], gsem[b]).wait()

    def scat_start(p, b):
        pltpu.async_copy(rows.at[b], acc.at[didx[p][b]], ssem[b], add=True)

    def scat_wait(p, b):
        pltpu.make_async_copy(rows.at[b], acc.at[didx[p][b]], ssem[b]).wait()

    pltpu.sync_copy(src_hbm.at[pl.ds(base + _NGRP * _GROUP, _TAIL)],
                    sflat0.at[pl.ds(0, _TAIL)])
    pltpu.sync_copy(dst_hbm.at[pl.ds(base + _NGRP * _GROUP, _TAIL)], d00)
    pltpu.async_copy(
        x_hbm.at[sflat0.at[pl.ds(0, _TAIL)]], rows.at[0], gsem[0]).wait()
    pltpu.async_copy(rows.at[0], acc.at[d00], ssem[0], add=True).wait()

    load_idx_group(0, 0)
    wait_idx(0)
    spread_didx(0)
    for b in range(_NBUF):
        gather_start(0, b)
    load_idx_group(1, 1)

    def step(g, p):
        wait_idx(p)
        spread_didx(p)
        for b in range(_NBUF):
            gather_wait(b)
            scat_start(1 - p, b)
        for b in range(_NBUF):
            scat_wait(1 - p, b)
            gather_start(p, b)

        @pl.when(g + 1 < _NGRP)
        def _():
            load_idx_group(g + 1, 1 - p)

    def pair_body(i, carry):
        step(2 * i + 1, 1)
        step(2 * i + 2, 0)
        return carry

    lax.fori_loop(0, (_NGRP - 1) // 2, pair_body, 0)
    if (_NGRP - 1) % 2 == 1:
        step(_NGRP - 1, (_NGRP - 1) % 2)
    _last_p = (_NGRP - 1) % 2

    for b in range(_NBUF):
        gather_wait(b)
        scat_start(_last_p, b)
    for b in range(_NBUF):
        scat_wait(_last_p, b)
    plsc.subcore_barrier()

    half = ROWS_PER_SUB // 2
    pltpu.async_copy(
        acc.at[pl.ds(s * ROWS_PER_SUB, half)],
        out_hbm.at[c, pl.ds(s * ROWS_PER_SUB, half)], gsem[0])
    pltpu.async_copy(
        acc.at[pl.ds(s * ROWS_PER_SUB + half, half)],
        out_hbm.at[c, pl.ds(s * ROWS_PER_SUB + half, half)], gsem[1])
    pltpu.make_async_copy(
        acc.at[pl.ds(0, half)],
        out_hbm.at[c, pl.ds(0, half)], gsem[0]).wait()
    pltpu.make_async_copy(
        acc.at[pl.ds(0, half)],
        out_hbm.at[c, pl.ds(0, half)], gsem[1]).wait()


def _tc0_body(featT_ref, w0_ref, x0_ref):
    feat = featT_ref[...]
    cols = lax.broadcasted_iota(jnp.int32, (1, FEAT_PAD), 1)
    h = jnp.zeros((N, FEAT_PAD), jnp.float32)
    for j in range(8):
        h = h + (feat[j][:, None] == cols).astype(jnp.float32)
    x0_ref[...] = jnp.dot(h, w0_ref[...], preferred_element_type=jnp.float32)


def _tc0_call(featT, w0p):
    return pl.pallas_call(
        _tc0_body,
        out_shape=jax.ShapeDtypeStruct((N, DIM), jnp.float32),
    )(featT, w0p)


def _tc1_body(cs_ref, cd_ref, x0_ref, onorm_ref, inorm_ref, x0n_ref):
    cs = jnp.sum(cs_ref[...], axis=(0, 1))
    cd = jnp.sum(cd_ref[...], axis=(0, 1))
    onorm = lax.rsqrt(jnp.maximum(cs, 1.0))
    inorm = lax.rsqrt(jnp.maximum(cd, 1.0))
    onorm_ref[...] = onorm
    inorm_ref[...] = inorm
    x0n_ref[...] = x0_ref[...] * onorm[:, None]


def _tc1_call(cnt_src, cnt_dst, x0):
    return pl.pallas_call(
        _tc1_body,
        out_shape=[
            jax.ShapeDtypeStruct((N,), jnp.float32),
            jax.ShapeDtypeStruct((N,), jnp.float32),
            jax.ShapeDtypeStruct((N, DIM), jnp.float32),
        ],
    )(cnt_src, cnt_dst, x0)


def _tc2_body(aggp_ref, inorm_ref, onorm_ref, w1_ref, b0_ref, x1n_ref):
    agg = aggp_ref[0, :N, :] + aggp_ref[1, :N, :]
    h1 = jnp.maximum(agg * inorm_ref[...][:, None] + b0_ref[...][None, :], 0.0)
    x1 = jnp.dot(h1, w1_ref[...], preferred_element_type=jnp.float32)
    x1n_ref[...] = x1 * onorm_ref[...][:, None]


def _tc2_call(aggp, inorm, onorm, w1, b0):
    return pl.pallas_call(
        _tc2_body,
        out_shape=jax.ShapeDtypeStruct((N, DIM), jnp.float32),
    )(aggp, inorm, onorm, w1, b0)


def _tc3_body(aggp_ref, inorm_ref, b1_ref, gid_ref, out_ref):
    h2 = (aggp_ref[0, :N, :] + aggp_ref[1, :N, :]) * inorm_ref[...][:, None] + b1_ref[...][None, :]
    nrm = jnp.sqrt(jnp.sum(h2 * h2, axis=1))
    factor = jnp.sqrt(jnp.float32(DIM)) / jnp.mean(nrm)
    h = h2 * factor
    seg = (gid_ref[...][None, :]
           == lax.broadcasted_iota(jnp.int32, (NUM_GRAPHS, 1), 0)).astype(jnp.float32)
    out_ref[...] = jnp.dot(seg, h, preferred_element_type=jnp.float32)


def _tc3_call(aggp, inorm, b1, gid):
    return pl.pallas_call(
        _tc3_body,
        out_shape=jax.ShapeDtypeStruct((NUM_GRAPHS, DIM), jnp.float32),
    )(aggp, inorm, b1, gid)


def kernel(feature, edge_index, graph_ids, W0, b0, W1, b1):
    edges = edge_index.astype(jnp.int32)
    src = edges[0]
    dst = edges[1]
    featT = feature.astype(jnp.int32).T
    w0p = jnp.pad(W0, ((0, FEAT_PAD - FEAT_LEN), (0, 0)))
    cnt_src, cnt_dst = _deg_kernel(src, dst)
    x0 = _tc0_call(featT, w0p)
    onorm, inorm, x0n = _tc1_call(cnt_src, cnt_dst, x0)
    agg1p = _agg_kernel(x0n, src, dst)
    x1n = _tc2_call(agg1p, inorm, onorm, W1, b0)
    agg2p = _agg_kernel(x1n, src, dst)
    return _tc3_call(agg2p, inorm, b1, graph_ids)


# speedup vs baseline: 1.0145x; 1.0145x over previous
import functools

import jax
import jax.numpy as jnp
from jax import lax
from jax.experimental import pallas as pl
from jax.experimental.pallas import tpu as pltpu
from jax.experimental.pallas import tpu_sc as plsc

N = 10000
E = 320000
FEAT_LEN = 150
FEAT_PAD = 152
DIM = 128
NUM_GRAPHS = 64

NC = 2
NS = 16
NW = NC * NS
EPT = E // NW
NP = 10240
ROWS_PER_SUB = NP // NS

_mesh = plsc.VectorSubcoreMesh(core_axis_name="c", subcore_axis_name="s")


_DEG_CH = 2000


@functools.partial(
    pl.kernel,
    mesh=_mesh,
    out_type=[
        jax.ShapeDtypeStruct((NW, 1, N), jnp.float32),
        jax.ShapeDtypeStruct((NW, 1, N), jnp.float32),
    ],
    scratch_types=[
        pltpu.VMEM((_DEG_CH,), jnp.int32),
        pltpu.VMEM((_DEG_CH,), jnp.int32),
        pltpu.VMEM((_DEG_CH,), jnp.int32),
        pltpu.VMEM((_DEG_CH,), jnp.int32),
        pltpu.VMEM((N,), jnp.float32),
        pltpu.VMEM((N,), jnp.float32),
        pltpu.SemaphoreType.DMA,
    ],
    compiler_params=pltpu.CompilerParams(needs_layout_passes=False),
)
def _deg_kernel(src_hbm, dst_hbm, osrc_hbm, odst_hbm,
                s0, s1, d0, d1, scnt, dcnt, isem):
    c = lax.axis_index("c")
    s = lax.axis_index("s")
    wid = s * NC + c
    base = wid * EPT
    sbuf = (s0, s1)
    dbuf = (d0, d1)
    nchunk = EPT // _DEG_CH

    zeros16 = jnp.zeros((16,), jnp.float32)

    def zero_body(i, carry):
        for u in range(5):
            scnt[pl.ds((i * 5 + u) * 16, 16)] = zeros16
            dcnt[pl.ds((i * 5 + u) * 16, 16)] = zeros16
        return carry

    lax.fori_loop(0, N // 80, zero_body, 0)

    ones16 = jnp.ones((16,), jnp.float32)

    def load_chunk(k, p):
        off = base + k * _DEG_CH
        pltpu.async_copy(src_hbm.at[pl.ds(off, _DEG_CH)], sbuf[p], isem)
        pltpu.async_copy(dst_hbm.at[pl.ds(off, _DEG_CH)], dbuf[p], isem)

    def wait_chunk(p):
        pltpu.make_async_copy(src_hbm.at[pl.ds(0, _DEG_CH)], sbuf[p], isem).wait()
        pltpu.make_async_copy(dst_hbm.at[pl.ds(0, _DEG_CH)], dbuf[p], isem).wait()

    load_chunk(0, 0)
    for k in range(nchunk):
        p = k % 2
        wait_chunk(p)
        if k + 1 < nchunk:
            load_chunk(k + 1, 1 - p)

        def vec_body(i, inner):
            for u in range(5):
                si = sbuf[p][pl.ds((i * 5 + u) * 16, 16)]
                plsc.addupdate_scatter(scnt, [si], ones16)
                di = dbuf[p][pl.ds((i * 5 + u) * 16, 16)]
                plsc.addupdate_scatter(dcnt, [di], ones16)
            return inner

        lax.fori_loop(0, _DEG_CH // 80, vec_body, 0)

    pltpu.sync_copy(scnt, osrc_hbm.at[wid, 0])
    pltpu.sync_copy(dcnt, odst_hbm.at[wid, 0])


_AGG_CH = 80
_NBUF = 4
_GROUP = _NBUF * _AGG_CH
_NGRP = EPT // _GROUP
_TAIL = EPT - _NGRP * _GROUP


@functools.partial(
    pl.kernel,
    mesh=_mesh,
    out_type=jax.ShapeDtypeStruct((NC, NP, DIM), jnp.float32),
    scratch_types=[
        pltpu.VMEM((_GROUP,), jnp.int32),
        pltpu.VMEM((_GROUP,), jnp.int32),
        pltpu.VMEM((_GROUP,), jnp.int32),
        pltpu.VMEM((_GROUP,), jnp.int32),
    ] + [pltpu.VMEM((_AGG_CH,), jnp.int32) for _ in range(2 * _NBUF)]
    + [
        pltpu.VMEM((_NBUF, _AGG_CH, DIM), jnp.float32),
        pltpu.VMEM((32, DIM), jnp.float32),
        pltpu.VMEM_SHARED((NP, DIM), jnp.float32),
        pltpu.SemaphoreType.DMA,
    ]
    + [pltpu.SemaphoreType.DMA for _ in range(2 * _NBUF)],
    compiler_params=pltpu.CompilerParams(needs_layout_passes=False),
)
def _agg_kernel(x_hbm, src_hbm, dst_hbm, out_hbm, sflat0, sflat1,
                dflat0, dflat1,
                d00, d01, d02, d03, d10, d11, d12, d13,
                rows, zbuf, acc, isem,
                gsem0, gsem1, gsem2, gsem3, ssem0, ssem1, ssem2, ssem3):
    c = lax.axis_index("c")
    s = lax.axis_index("s")
    wid = s * NC + c
    base = wid * EPT
    sflat = (sflat0, sflat1)
    dflat = (dflat0, dflat1)
    didx = ((d00, d01, d02, d03), (d10, d11, d12, d13))
    gsem = (gsem0, gsem1, gsem2, gsem3)
    ssem = (ssem0, ssem1, ssem2, ssem3)

    zeros16 = jnp.zeros((16,), jnp.float32)

    def zfill(i, carry):
        def zlane(j, inner):
            zbuf[i, pl.ds(j * 16, 16)] = zeros16
            return inner

        return lax.fori_loop(0, DIM // 16, zlane, carry)

    lax.fori_loop(0, 32, zfill, 0)

    def zcopy(t, carry):
        pltpu.async_copy(zbuf, acc.at[pl.ds(s * ROWS_PER_SUB + t * 32, 32)], isem)
        return carry

    lax.fori_loop(0, ROWS_PER_SUB // 32, zcopy, 0)

    def zdrain(t, carry):
        pltpu.make_async_copy(zbuf, acc.at[pl.ds(0, 32)], isem).wait()
        return carry

    lax.fori_loop(0, ROWS_PER_SUB // 32, zdrain, 0)
    plsc.subcore_barrier()

    def load_idx_group(g, p):
        off = base + g * _GROUP
        pltpu.async_copy(src_hbm.at[pl.ds(off, _GROUP)], sflat[p], isem)
        pltpu.async_copy(dst_hbm.at[pl.ds(off, _GROUP)], dflat[p], isem)

    def wait_idx(p):
        pltpu.make_async_copy(src_hbm.at[pl.ds(0, _GROUP)], sflat[p], isem).wait()
        pltpu.make_async_copy(dst_hbm.at[pl.ds(0, _GROUP)], dflat[p], isem).wait()

    def spread_didx(p):
        for b in range(_NBUF):
            for j in range(_AGG_CH // 16):
                didx[p][b][pl.ds(j * 16, 16)] = (
                    dflat[p][pl.ds(b * _AGG_CH + j * 16, 16)])

    def gather_start(p, b):
        pltpu.async_copy(
            x_hbm.at[sflat[p].at[pl.ds(b * _AGG_CH, _AGG_CH)]], rows.at[b], gsem[b])

    def gather_wait(b):
        pltpu.make_async_copy(
            x_hbm.at[sflat[0].at[pl.ds(0, _AGG_CH)]], rows.at[b], gsem[b]).wait()

    def scat_start(p, b):
        pltpu.async_copy(rows.at[b], acc.at[didx[p][b]], ssem[b], add=True)

    def scat_wait(p, b):
        pltpu.make_async_copy(rows.at[b], acc.at[didx[p][b]], ssem[b]).wait()

    pltpu.sync_copy(src_hbm.at[pl.ds(base + _NGRP * _GROUP, _TAIL)],
                    sflat0.at[pl.ds(0, _TAIL)])
    pltpu.sync_copy(dst_hbm.at[pl.ds(base + _NGRP * _GROUP, _TAIL)], d00)
    pltpu.async_copy(
        x_hbm.at[sflat0.at[pl.ds(0, _TAIL)]], rows.at[0], gsem[0]).wait()
    pltpu.async_copy(rows.at[0], acc.at[d00], ssem[0], add=True).wait()

    load_idx_group(0, 0)
    wait_idx(0)
    spread_didx(0)
    for b in range(_NBUF):
        gather_start(0, b)
    load_idx_group(1, 1)

    def step(g, p):
        wait_idx(p)
        spread_didx(p)
        for b in range(_NBUF):
            gather_wait(b)
            scat_start(1 - p, b)
        for b in range(_NBUF):
            scat_wait(1 - p, b)
            gather_start(p, b)

        @pl.when(g + 1 < _NGRP)
        def _():
            load_idx_group(g + 1, 1 - p)

    def pair_body(i, carry):
        step(2 * i + 1, 1)
        step(2 * i + 2, 0)
        return carry

    lax.fori_loop(0, (_NGRP - 1) // 2, pair_body, 0)
    if (_NGRP - 1) % 2 == 1:
        step(_NGRP - 1, (_NGRP - 1) % 2)
    _last_p = (_NGRP - 1) % 2

    for b in range(_NBUF):
        gather_wait(b)
        scat_start(_last_p, b)
    for b in range(_NBUF):
        scat_wait(_last_p, b)
    plsc.subcore_barrier()

    half = ROWS_PER_SUB // 2
    pltpu.async_copy(
        acc.at[pl.ds(s * ROWS_PER_SUB, half)],
        out_hbm.at[c, pl.ds(s * ROWS_PER_SUB, half)], gsem[0])
    pltpu.async_copy(
        acc.at[pl.ds(s * ROWS_PER_SUB + half, half)],
        out_hbm.at[c, pl.ds(s * ROWS_PER_SUB + half, half)], gsem[1])
    pltpu.make_async_copy(
        acc.at[pl.ds(0, half)],
        out_hbm.at[c, pl.ds(0, half)], gsem[0]).wait()
    pltpu.make_async_copy(
        acc.at[pl.ds(0, half)],
        out_hbm.at[c, pl.ds(0, half)], gsem[1]).wait()


def _tc0_body(featT_ref, w0_ref, x0_ref):
    feat = featT_ref[...]
    cols = lax.broadcasted_iota(jnp.int32, (1, FEAT_PAD), 1)
    h = jnp.zeros((N, FEAT_PAD), jnp.float32)
    for j in range(8):
        h = h + (feat[j][:, None] == cols).astype(jnp.float32)
    x0_ref[...] = jnp.dot(h, w0_ref[...], preferred_element_type=jnp.float32)


def _tc0_call(featT, w0p):
    return pl.pallas_call(
        _tc0_body,
        out_shape=jax.ShapeDtypeStruct((N, DIM), jnp.float32),
    )(featT, w0p)


def _tc1_body(cs_ref, cd_ref, x0_ref, onorm_ref, inorm_ref, x0n_ref):
    cs = jnp.sum(cs_ref[...], axis=(0, 1))
    cd = jnp.sum(cd_ref[...], axis=(0, 1))
    onorm = lax.rsqrt(jnp.maximum(cs, 1.0))
    inorm = lax.rsqrt(jnp.maximum(cd, 1.0))
    onorm_ref[...] = onorm
    inorm_ref[...] = inorm
    x0n_ref[...] = x0_ref[...] * onorm[:, None]


def _tc1_call(cnt_src, cnt_dst, x0):
    return pl.pallas_call(
        _tc1_body,
        out_shape=[
            jax.ShapeDtypeStruct((N,), jnp.float32),
            jax.ShapeDtypeStruct((N,), jnp.float32),
            jax.ShapeDtypeStruct((N, DIM), jnp.float32),
        ],
    )(cnt_src, cnt_dst, x0)


def _tc2_body(aggp_ref, inorm_ref, onorm_ref, w1_ref, b0_ref, x1n_ref):
    agg = aggp_ref[0, :N, :] + aggp_ref[1, :N, :]
    h1 = jnp.maximum(agg * inorm_ref[...][:, None] + b0_ref[...][None, :], 0.0)
    x1 = jnp.dot(h1, w1_ref[...], preferred_element_type=jnp.float32)
    x1n_ref[...] = x1 * onorm_ref[...][:, None]


def _tc2_call(aggp, inorm, onorm, w1, b0):
    return pl.pallas_call(
        _tc2_body,
        out_shape=jax.ShapeDtypeStruct((N, DIM), jnp.float32),
    )(aggp, inorm, onorm, w1, b0)


def _tc3_body(aggp_ref, inorm_ref, b1_ref, gid_ref, out_ref):
    h2 = (aggp_ref[0, :N, :] + aggp_ref[1, :N, :]) * inorm_ref[...][:, None] + b1_ref[...][None, :]
    nrm = jnp.sqrt(jnp.sum(h2 * h2, axis=1))
    factor = jnp.sqrt(jnp.float32(DIM)) / jnp.mean(nrm)
    h = h2 * factor
    seg = (gid_ref[...][None, :]
           == lax.broadcasted_iota(jnp.int32, (NUM_GRAPHS, 1), 0)).astype(jnp.float32)
    out_ref[...] = jnp.dot(seg, h, preferred_element_type=jnp.float32)


def _tc3_call(aggp, inorm, b1, gid):
    return pl.pallas_call(
        _tc3_body,
        out_shape=jax.ShapeDtypeStruct((NUM_GRAPHS, DIM), jnp.float32),
    )(aggp, inorm, b1, gid)


def kernel(feature, edge_index, graph_ids, W0, b0, W1, b1):
    edges = edge_index.astype(jnp.int32)
    src = edges[0]
    dst = edges[1]
    featT = feature.astype(jnp.int32).T
    w0p = jnp.pad(W0, ((0, FEAT_PAD - FEAT_LEN), (0, 0)))
    cnt_src, cnt_dst = _deg_kernel(src, dst)
    x0 = _tc0_call(featT, w0p)
    onorm, inorm, x0n = _tc1_call(cnt_src, cnt_dst, x0)
    agg1p = _agg_kernel(x0n, src, dst)
    x1n = _tc2_call(agg1p, inorm, onorm, W1, b0)
    agg2p = _agg_kernel(x1n, src, dst)
    return _tc3_call(agg2p, inorm, b1, graph_ids)
